# baseline (speedup 1.0000x reference)
import functools

import jax
import jax.numpy as jnp
from jax import lax
from jax.experimental import pallas as pl
from jax.experimental.pallas import tpu as pltpu
from jax.experimental.pallas import tpu_sc as plsc

B = 16384
D = 32
NC = 2
NS = 16
NW = NC * NS
BPW = B // NW
GCH = 128
NCH = BPW // GCH
L = 16


def _sc_scores_body(idx_hbm, uw_hbm, iw_hbm, out_hbm,
                    idx_u, idx_i, idx_j, ru, ri, rj, out_v, sem):
    wid = lax.axis_index("s") * NC + lax.axis_index("c")
    r0 = wid * NCH

    pltpu.sync_copy(idx_hbm.at[pl.ds(r0, NCH)], idx_u)
    pltpu.sync_copy(idx_hbm.at[pl.ds(B // GCH + r0, NCH)], idx_i)
    pltpu.sync_copy(idx_hbm.at[pl.ds(2 * (B // GCH) + r0, NCH)], idx_j)

    cps = []
    for c in range(NCH):
        dst = pl.ds(c * GCH, GCH)
        cps.append(pltpu.async_copy(uw_hbm.at[idx_u.at[c]], ru.at[dst], sem))
        cps.append(pltpu.async_copy(iw_hbm.at[idx_i.at[c]], ri.at[dst], sem))
        cps.append(pltpu.async_copy(iw_hbm.at[idx_j.at[c]], rj.at[dst], sem))
    for cp in cps:
        cp.wait()

    lanes = lax.iota(jnp.int32, L)

    @pl.loop(0, BPW // L)
    def _(g):
        rows = g * L + lanes
        acc = jnp.zeros((L,), jnp.float32)
        for d in range(D):
            cols = jnp.full((L,), d, jnp.int32)
            uu = plsc.load_gather(ru, [rows, cols])
            vi = plsc.load_gather(ri, [rows, cols])
            vj = plsc.load_gather(rj, [rows, cols])
            acc = acc + uu * (vi - vj)
        out_v[pl.ds(g * L, L)] = acc

    pltpu.sync_copy(out_v, out_hbm.at[pl.ds(wid * BPW, BPW)])


def _tc_loss_body(z_ref, o_ref):
    z = z_ref[...]
    t = -z
    sp = jnp.maximum(t, 0.0) + jnp.log(1.0 + jnp.exp(-jnp.abs(t)))
    o_ref[0, 0] = jnp.sum(sp) * (1.0 / B)


def kernel(x, user_weight, item_weight):
    idx = x.astype(jnp.int32).T.reshape(3 * B // GCH, GCH)

    mesh = plsc.VectorSubcoreMesh(core_axis_name="c", subcore_axis_name="s")
    sc_scores = pl.kernel(
        _sc_scores_body,
        out_type=jax.ShapeDtypeStruct((B,), jnp.float32),
        mesh=mesh,
        compiler_params=pltpu.CompilerParams(
            needs_layout_passes=False, use_tc_tiling_on_sc=False),
        scratch_types=[
            pltpu.VMEM((NCH, GCH), jnp.int32),
            pltpu.VMEM((NCH, GCH), jnp.int32),
            pltpu.VMEM((NCH, GCH), jnp.int32),
            pltpu.VMEM((BPW, D), jnp.float32),
            pltpu.VMEM((BPW, D), jnp.float32),
            pltpu.VMEM((BPW, D), jnp.float32),
            pltpu.VMEM((BPW,), jnp.float32),
            pltpu.SemaphoreType.DMA,
        ],
    )
    xuij = sc_scores(idx, user_weight, item_weight)

    loss = pl.pallas_call(
        _tc_loss_body,
        out_shape=jax.ShapeDtypeStruct((1, 1), jnp.float32),
        in_specs=[pl.BlockSpec(memory_space=pltpu.VMEM)],
        out_specs=pl.BlockSpec(memory_space=pltpu.SMEM),
    )(xuij.reshape(B // 128, 128))
    return loss[0, 0]
